# baseline (speedup 1.0000x reference)
import functools

import jax
import jax.numpy as jnp
from jax import lax
from jax.experimental import pallas as pl
from jax.experimental.pallas import tpu as pltpu
from jax.experimental.pallas import tpu_sc as plsc

N = 10000
E = 160000
D_IN = 256
H = 512
C = 128
L = 4
EPS = 1e-5

R = 160
NCHUNK = 64
NPAD = NCHUNK * R
REGION = 168
TRASH = 160
SBSIZE = 4096
BSTRIDE = E + SBSIZE
EP = 16000
NPIECE = E // EP
B = 32
RB = 256

_SC_MESH = dict(core_axis_name="c", subcore_axis_name="s")


@functools.partial(
    pl.kernel,
    out_type=(
        jax.ShapeDtypeStruct((NPAD,), jnp.float32),
        jax.ShapeDtypeStruct((NCHUNK * BSTRIDE,), jnp.int32),
        jax.ShapeDtypeStruct((NCHUNK * 16,), jnp.int32),
    ),
    mesh=plsc.VectorSubcoreMesh(**_SC_MESH),
    compiler_params=pltpu.CompilerParams(needs_layout_passes=False),
    scratch_types=[
        pltpu.VMEM((EP,), jnp.int32),
        pltpu.VMEM((EP,), jnp.int32),
        pltpu.VMEM((SBSIZE + 16,), jnp.int32),
        pltpu.VMEM((R,), jnp.float32),
        pltpu.VMEM((16,), jnp.int32),
    ],
)
def _sc_bucketize(ei, deg, buckets, counts, src_v, dst_v, sb, hist, cbuf):
    w = lax.axis_index("s") * 2 + lax.axis_index("c")
    zero16 = jnp.zeros((16,), jnp.float32)
    for sub in range(2):
        c = w * 2 + sub
        base = c * R
        for j in range(R // 16):
            hist[pl.ds(j * 16, 16)] = zero16

        def piece_body(p, carry):
            pltpu.sync_copy(ei.at[pl.ds(pl.multiple_of(p * EP, 8), EP)],
                            src_v)
            pltpu.sync_copy(ei.at[pl.ds(pl.multiple_of(E + p * EP, 8), EP)],
                            dst_v)

            def ebody(i, carry):
                cnt, off = carry
                s16 = src_v[pl.ds(i * 16, 16)]
                d16 = dst_v[pl.ds(i * 16, 16)]
                ld = d16 - base
                m = (ld >= 0) & (ld < R)
                lds = jnp.where(m, ld, 0)
                plsc.addupdate_scatter(
                    hist, [lds], jnp.where(m, 1.0, 0.0), mask=m)
                packed = s16 | (lds << 14)
                plsc.store_compressed(sb.at[pl.ds(cnt, 16)], packed, mask=m)
                cnt = cnt + jnp.sum(m.astype(jnp.int32))

                def do_flush(a):
                    cnt, off = a
                    pltpu.sync_copy(
                        sb.at[pl.ds(0, SBSIZE)],
                        buckets.at[pl.ds(
                            pl.multiple_of(c * BSTRIDE + off, 8), SBSIZE)])
                    spill = sb[pl.ds(SBSIZE, 16)]
                    sb[pl.ds(0, 16)] = spill
                    return cnt - SBSIZE, off + SBSIZE

                return lax.cond(cnt >= SBSIZE, do_flush, lambda a: a,
                                (cnt, off))

            return lax.fori_loop(0, EP // 16, ebody, carry)

        cnt, off = lax.fori_loop(0, NPIECE, piece_body,
                                 (jnp.int32(0), jnp.int32(0)))
        pltpu.sync_copy(sb.at[pl.ds(0, SBSIZE)],
                        buckets.at[pl.ds(
                            pl.multiple_of(c * BSTRIDE + off, 8), SBSIZE)])
        cbuf[pl.ds(0, 16)] = jnp.full((16,), cnt + off, jnp.int32)
        pltpu.sync_copy(cbuf, counts.at[pl.ds(pl.multiple_of(c * 16, 8), 16)])
        pltpu.sync_copy(hist, deg.at[pl.ds(pl.multiple_of(base, 8), R)])


@functools.partial(
    pl.kernel,
    out_type=jax.ShapeDtypeStruct((NPAD, H), jnp.float32),
    mesh=plsc.VectorSubcoreMesh(**_SC_MESH),
    scratch_types=[
        pltpu.VMEM((B,), jnp.int32),
        pltpu.VMEM((B,), jnp.int32),
        pltpu.VMEM((B,), jnp.int32),
        pltpu.VMEM((B,), jnp.int32),
        pltpu.VMEM((B,), jnp.int32),
        pltpu.VMEM((B, H), jnp.float32),
        pltpu.VMEM((B, H), jnp.float32),
        pltpu.VMEM((16,), jnp.int32),
        pltpu.VMEM((REGION, H), jnp.float32),
        pltpu.SemaphoreType.DMA,
    ],
)
def _sc_aggregate(zs, buckets, counts, agg,
                  pk, sidx0, lidx0, sidx1, lidx1, gbuf0, gbuf1, cbuf, acc,
                  sem):
    sid = lax.axis_index("s")
    w = sid * 2 + lax.axis_index("c")
    z16 = jnp.zeros((16,), jnp.float32)

    for r in range(2):
        c = w * 2 + r

        def zrow(j, _):
            for k in range(H // 16):
                acc[j, pl.ds(k * 16, 16)] = z16
            return 0

        lax.fori_loop(0, REGION, zrow, 0)
        pltpu.sync_copy(counts.at[pl.ds(pl.multiple_of(c * 16, 8), 16)], cbuf)
        total = cbuf[pl.ds(0, 16)][0]
        nb = (total + (B - 1)) // B

        def build(b, sidx, lidx):
            pltpu.sync_copy(
                buckets.at[pl.ds(
                    pl.multiple_of(c * BSTRIDE + b * B, 8), B)], pk)
            for g in range(B // 16):
                pv = pk[pl.ds(g * 16, 16)]
                pos = b * B + g * 16 + lax.iota(jnp.int32, 16)
                valid = pos < total
                sidx[pl.ds(g * 16, 16)] = jnp.where(valid, pv & 0x3FFF, 0)
                lidx[pl.ds(g * 16, 16)] = jnp.where(
                    valid, lax.shift_right_logical(pv, 14), TRASH)

        def fire(sidx, gbuf):
            pltpu.async_copy(zs.at[sidx], gbuf, sem)

        def drain(sidx, gbuf):
            pltpu.make_async_copy(zs.at[sidx], gbuf, sem).wait()

        def accum(lidx, gbuf):
            for g in range(B // 16):
                lv = lidx[pl.ds(g * 16, 16)]
                lds = [lv[t] for t in range(16)]

                def cols(k, _):
                    for t in range(16):
                        for cc in range(8):
                            co = pl.ds(
                                pl.multiple_of(k * 128 + cc * 16, 16), 16)
                            acc[lds[t], co] = acc[lds[t], co] \
                                + gbuf[g * 16 + t, co]
                    return 0

                lax.fori_loop(0, H // 128, cols, 0)

        @pl.when(nb > 0)
        def _():
            build(0, sidx0, lidx0)
            fire(sidx0, gbuf0)

        def pair(p, _):
            b1 = 2 * p + 1
            b2 = 2 * p + 2

            @pl.when(b1 < nb)
            def _():
                build(b1, sidx1, lidx1)
                fire(sidx1, gbuf1)

            drain(sidx0, gbuf0)
            accum(lidx0, gbuf0)

            @pl.when(b2 < nb)
            def _():
                build(b2, sidx0, lidx0)
                fire(sidx0, gbuf0)

            @pl.when(b1 < nb)
            def _():
                drain(sidx1, gbuf1)
                accum(lidx1, gbuf1)

            return 0

        lax.fori_loop(0, (nb + 1) // 2, pair, 0)
        pltpu.sync_copy(acc.at[pl.ds(0, R)],
                        agg.at[pl.ds(pl.multiple_of(c * R, 8), R)])


def _gelu(x):
    return 0.5 * x * (1.0 + lax.erf(x * 0.7071067811865476))


def _ln(x, g, b):
    mu = jnp.mean(x, axis=-1, keepdims=True)
    var = jnp.mean((x - mu) ** 2, axis=-1, keepdims=True)
    return (x - mu) * lax.rsqrt(var + EPS) * g + b


def _tc_init_body(x_ref, w_ref, b_ref, deg_ref, h_ref, dis_ref):
    h = jnp.dot(x_ref[...], w_ref[...],
                preferred_element_type=jnp.float32) + b_ref[...]
    h_ref[...] = _gelu(h)
    dis_ref[...] = lax.rsqrt(deg_ref[...] + 1.0)


def _tc_layer_body(first, h_or_agg_ref, zsp_ref, hresp_ref, dis_ref, cb_ref,
                   g_ref, bt_ref, w_ref, zs_ref, hres_ref):
    dis = jnp.reshape(dis_ref[...], (RB, 1))
    if first:
        h = h_or_agg_ref[...]
    else:
        h = dis * (h_or_agg_ref[...] + zsp_ref[...]) + cb_ref[...] \
            + hresp_ref[...]
    hres_ref[...] = _ln(h, g_ref[...], bt_ref[...])
    z = jnp.dot(h, w_ref[...], preferred_element_type=jnp.float32)
    zs_ref[...] = z * dis


def _tc_final_body(agg_ref, zsp_ref, hresp_ref, dis_ref, cb_ref,
                   w1_ref, b1_ref, g_ref, bt_ref, w2_ref, b2_ref, out_ref):
    dis = jnp.reshape(dis_ref[...], (RB, 1))
    h = dis * (agg_ref[...] + zsp_ref[...]) + cb_ref[...] + hresp_ref[...]
    z = jnp.dot(h, w1_ref[...], preferred_element_type=jnp.float32) \
        + b1_ref[...]
    z = _gelu(z)
    z = _ln(z, g_ref[...], bt_ref[...])
    o = jnp.dot(z, w2_ref[...], preferred_element_type=jnp.float32) \
        + b2_ref[...]
    m = jnp.max(o, axis=-1, keepdims=True)
    out_ref[...] = o - m - jnp.log(
        jnp.sum(jnp.exp(o - m), axis=-1, keepdims=True))


_GRID = NPAD // RB


def _row_spec(cols):
    if cols is None:
        return pl.BlockSpec((RB,), lambda i: (i,))
    return pl.BlockSpec((RB, cols), lambda i: (i, 0))


def _full_spec(shape):
    if len(shape) == 1:
        return pl.BlockSpec(shape, lambda i: (0,))
    return pl.BlockSpec(shape, lambda i: (0, 0))


def _tc_init(x, W_init, b_init, deg):
    return pl.pallas_call(
        _tc_init_body,
        grid=(_GRID,),
        in_specs=[_row_spec(D_IN), _full_spec((D_IN, H)), _full_spec((H,)),
                  _row_spec(None)],
        out_specs=[_row_spec(H), _row_spec(None)],
        out_shape=(jax.ShapeDtypeStruct((NPAD, H), jnp.float32),
                   jax.ShapeDtypeStruct((NPAD,), jnp.float32)),
    )(x, W_init, b_init, deg)


def _tc_layer(first, h_or_agg, zsp, hresp, dis, cb, g, bt, w):
    return pl.pallas_call(
        functools.partial(_tc_layer_body, first),
        grid=(_GRID,),
        in_specs=[_row_spec(H), _row_spec(H), _row_spec(H), _row_spec(None),
                  _full_spec((H,)), _full_spec((H,)), _full_spec((H,)),
                  _full_spec((H, H))],
        out_specs=[_row_spec(H), _row_spec(H)],
        out_shape=(jax.ShapeDtypeStruct((NPAD, H), jnp.float32),
                   jax.ShapeDtypeStruct((NPAD, H), jnp.float32)),
    )(h_or_agg, zsp, hresp, dis, cb, g, bt, w)


def _tc_final(agg, zsp, hresp, dis, cb, W_m1, b_m1, g, bt, W_m2, b_m2):
    return pl.pallas_call(
        _tc_final_body,
        grid=(_GRID,),
        in_specs=[_row_spec(H), _row_spec(H), _row_spec(H), _row_spec(None),
                  _full_spec((H,)), _full_spec((H, H)), _full_spec((H,)),
                  _full_spec((H,)), _full_spec((H,)), _full_spec((H, C)),
                  _full_spec((C,))],
        out_specs=[_row_spec(C)],
        out_shape=(jax.ShapeDtypeStruct((NPAD, C), jnp.float32),),
    )(agg, zsp, hresp, dis, cb, W_m1, b_m1, g, bt, W_m2, b_m2)


def kernel(x, edge_index, W_init, b_init, ln_gamma, ln_beta, conv_W, conv_b,
           W_m1, b_m1, mlp_ln_g, mlp_ln_b, W_m2, b_m2):
    deg, buckets, counts = _sc_bucketize(jnp.ravel(edge_index))
    xp = jnp.zeros((NPAD, D_IN), jnp.float32).at[:N].set(x)
    h0, dis = _tc_init(xp, W_init, b_init, deg)

    zs, hres = _tc_layer(True, h0, h0, h0, dis, conv_b[0],
                         ln_gamma[0], ln_beta[0], conv_W[0])
    for i in range(1, L):
        agg = _sc_aggregate(zs, buckets, counts)
        zs, hres = _tc_layer(False, agg, zs, hres, dis, conv_b[i - 1],
                             ln_gamma[i], ln_beta[i], conv_W[i])
    agg = _sc_aggregate(zs, buckets, counts)
    (out,) = _tc_final(agg, zs, hres, dis, conv_b[L - 1],
                       W_m1, b_m1, mlp_ln_g, mlp_ln_b, W_m2, b_m2)
    return out[:N]


# speedup vs baseline: 1.0611x; 1.0611x over previous
import functools

import jax
import jax.numpy as jnp
from jax import lax
from jax.experimental import pallas as pl
from jax.experimental.pallas import tpu as pltpu
from jax.experimental.pallas import tpu_sc as plsc

N = 10000
E = 160000
D_IN = 256
H = 512
C = 128
L = 4
EPS = 1e-5

R = 160
NCHUNK = 64
NPAD = NCHUNK * R
REGION = 168
TRASH = 160
SBSIZE = 4096
BSTRIDE = E + SBSIZE
EP = 16000
NPIECE = E // EP
B = 32
RB = 256

_SC_MESH = dict(core_axis_name="c", subcore_axis_name="s")


@functools.partial(
    pl.kernel,
    out_type=(
        jax.ShapeDtypeStruct((NPAD,), jnp.float32),
        jax.ShapeDtypeStruct((NCHUNK * BSTRIDE,), jnp.int32),
        jax.ShapeDtypeStruct((NCHUNK * 16,), jnp.int32),
    ),
    mesh=plsc.VectorSubcoreMesh(**_SC_MESH),
    compiler_params=pltpu.CompilerParams(needs_layout_passes=False),
    scratch_types=[
        pltpu.VMEM((EP,), jnp.int32),
        pltpu.VMEM((EP,), jnp.int32),
        pltpu.VMEM((SBSIZE + 16,), jnp.int32),
        pltpu.VMEM((R,), jnp.float32),
        pltpu.VMEM((16,), jnp.int32),
    ],
)
def _sc_bucketize(ei, deg, buckets, counts, src_v, dst_v, sb, hist, cbuf):
    w = lax.axis_index("s") * 2 + lax.axis_index("c")
    zero16 = jnp.zeros((16,), jnp.float32)
    for sub in range(2):
        c = w * 2 + sub
        base = c * R
        for j in range(R // 16):
            hist[pl.ds(j * 16, 16)] = zero16

        def piece_body(p, carry):
            pltpu.sync_copy(ei.at[pl.ds(pl.multiple_of(p * EP, 8), EP)],
                            src_v)
            pltpu.sync_copy(ei.at[pl.ds(pl.multiple_of(E + p * EP, 8), EP)],
                            dst_v)

            def ebody(i, carry):
                cnt, off = carry
                s16 = src_v[pl.ds(i * 16, 16)]
                d16 = dst_v[pl.ds(i * 16, 16)]
                ld = d16 - base
                m = (ld >= 0) & (ld < R)
                lds = jnp.where(m, ld, 0)
                plsc.addupdate_scatter(
                    hist, [lds], jnp.where(m, 1.0, 0.0), mask=m)
                packed = s16 | (lds << 14)
                plsc.store_compressed(sb.at[pl.ds(cnt, 16)], packed, mask=m)
                cnt = cnt + jnp.sum(m.astype(jnp.int32))

                def do_flush(a):
                    cnt, off = a
                    pltpu.sync_copy(
                        sb.at[pl.ds(0, SBSIZE)],
                        buckets.at[pl.ds(
                            pl.multiple_of(c * BSTRIDE + off, 8), SBSIZE)])
                    spill = sb[pl.ds(SBSIZE, 16)]
                    sb[pl.ds(0, 16)] = spill
                    return cnt - SBSIZE, off + SBSIZE

                return lax.cond(cnt >= SBSIZE, do_flush, lambda a: a,
                                (cnt, off))

            return lax.fori_loop(0, EP // 16, ebody, carry)

        cnt, off = lax.fori_loop(0, NPIECE, piece_body,
                                 (jnp.int32(0), jnp.int32(0)))
        pltpu.sync_copy(sb.at[pl.ds(0, SBSIZE)],
                        buckets.at[pl.ds(
                            pl.multiple_of(c * BSTRIDE + off, 8), SBSIZE)])
        cbuf[pl.ds(0, 16)] = jnp.full((16,), cnt + off, jnp.int32)
        pltpu.sync_copy(cbuf, counts.at[pl.ds(pl.multiple_of(c * 16, 8), 16)])
        pltpu.sync_copy(hist, deg.at[pl.ds(pl.multiple_of(base, 8), R)])


@functools.partial(
    pl.kernel,
    out_type=jax.ShapeDtypeStruct((NPAD, H), jnp.float32),
    mesh=plsc.VectorSubcoreMesh(**_SC_MESH),
    scratch_types=[
        pltpu.VMEM((4096,), jnp.int32),
        pltpu.VMEM((B,), jnp.int32),
        pltpu.VMEM((B,), jnp.int32),
        pltpu.VMEM((B,), jnp.int32),
        pltpu.VMEM((B,), jnp.int32),
        pltpu.VMEM((B, H), jnp.float32),
        pltpu.VMEM((B, H), jnp.float32),
        pltpu.VMEM((16,), jnp.int32),
        pltpu.VMEM((REGION, H), jnp.float32),
        pltpu.SemaphoreType.DMA,
    ],
)
def _sc_aggregate(zs, buckets, counts, agg,
                  pk, sidx0, lidx0, sidx1, lidx1, gbuf0, gbuf1, cbuf, acc,
                  sem):
    sid = lax.axis_index("s")
    w = sid * 2 + lax.axis_index("c")
    z16 = jnp.zeros((16,), jnp.float32)

    for r in range(2):
        c = w * 2 + r

        def zrow(j, _):
            for k in range(H // 16):
                acc[j, pl.ds(k * 16, 16)] = z16
            return 0

        lax.fori_loop(0, REGION, zrow, 0)
        pltpu.sync_copy(counts.at[pl.ds(pl.multiple_of(c * 16, 8), 16)], cbuf)
        total = cbuf[pl.ds(0, 16)][0]
        nb = (total + (B - 1)) // B

        PKB = 4096
        BPP = PKB // B
        npc = (total + (PKB - 1)) // PKB

        def build(q, bq, sidx, lidx):
            for g in range(B // 16):
                pv = pk[pl.ds(pl.multiple_of(bq * B + g * 16, 16), 16)]
                pos = q * PKB + bq * B + g * 16 + lax.iota(jnp.int32, 16)
                valid = pos < total
                sidx[pl.ds(g * 16, 16)] = jnp.where(valid, pv & 0x3FFF, 0)
                lidx[pl.ds(g * 16, 16)] = jnp.where(
                    valid, lax.shift_right_logical(pv, 14), TRASH)

        def fire(sidx, gbuf):
            pltpu.async_copy(zs.at[sidx], gbuf, sem)

        def drain(sidx, gbuf):
            pltpu.make_async_copy(zs.at[sidx], gbuf, sem).wait()

        def accum(lidx, gbuf):
            for g in range(B // 16):
                lv = lidx[pl.ds(g * 16, 16)]
                lds = [lv[t] for t in range(16)]

                def cols(k, _):
                    for t in range(16):
                        for cc in range(4):
                            co = pl.ds(
                                pl.multiple_of(k * 64 + cc * 16, 16), 16)
                            acc[lds[t], co] = acc[lds[t], co] \
                                + gbuf[g * 16 + t, co]
                    return 0

                lax.fori_loop(0, H // 64, cols, 0)

        def piece(q, _):
            pltpu.sync_copy(
                buckets.at[pl.ds(
                    pl.multiple_of(c * BSTRIDE + q * PKB, 8), PKB)], pk)
            nbq = jnp.minimum(nb - q * BPP, BPP)
            build(q, 0, sidx0, lidx0)
            fire(sidx0, gbuf0)

            def pair(p, _):
                b1 = 2 * p + 1
                b2 = 2 * p + 2

                @pl.when(b1 < nbq)
                def _():
                    build(q, b1, sidx1, lidx1)
                    fire(sidx1, gbuf1)

                drain(sidx0, gbuf0)
                accum(lidx0, gbuf0)

                @pl.when(b2 < nbq)
                def _():
                    build(q, b2, sidx0, lidx0)
                    fire(sidx0, gbuf0)

                @pl.when(b1 < nbq)
                def _():
                    drain(sidx1, gbuf1)
                    accum(lidx1, gbuf1)

                return 0

            lax.fori_loop(0, (nbq + 1) // 2, pair, 0)
            return 0

        lax.fori_loop(0, npc, piece, 0)
        pltpu.sync_copy(acc.at[pl.ds(0, R)],
                        agg.at[pl.ds(pl.multiple_of(c * R, 8), R)])


def _gelu(x):
    return 0.5 * x * (1.0 + lax.erf(x * 0.7071067811865476))


def _ln(x, g, b):
    mu = jnp.mean(x, axis=-1, keepdims=True)
    var = jnp.mean((x - mu) ** 2, axis=-1, keepdims=True)
    return (x - mu) * lax.rsqrt(var + EPS) * g + b


def _tc_init_body(x_ref, w_ref, b_ref, deg_ref, h_ref, dis_ref):
    h = jnp.dot(x_ref[...], w_ref[...],
                preferred_element_type=jnp.float32) + b_ref[...]
    h_ref[...] = _gelu(h)
    dis_ref[...] = lax.rsqrt(deg_ref[...] + 1.0)


def _tc_layer_body(first, h_or_agg_ref, zsp_ref, hresp_ref, dis_ref, cb_ref,
                   g_ref, bt_ref, w_ref, zs_ref, hres_ref):
    dis = jnp.reshape(dis_ref[...], (RB, 1))
    if first:
        h = h_or_agg_ref[...]
    else:
        h = dis * (h_or_agg_ref[...] + zsp_ref[...]) + cb_ref[...] \
            + hresp_ref[...]
    hres_ref[...] = _ln(h, g_ref[...], bt_ref[...])
    z = jnp.dot(h, w_ref[...], preferred_element_type=jnp.float32)
    zs_ref[...] = z * dis


def _tc_final_body(agg_ref, zsp_ref, hresp_ref, dis_ref, cb_ref,
                   w1_ref, b1_ref, g_ref, bt_ref, w2_ref, b2_ref, out_ref):
    dis = jnp.reshape(dis_ref[...], (RB, 1))
    h = dis * (agg_ref[...] + zsp_ref[...]) + cb_ref[...] + hresp_ref[...]
    z = jnp.dot(h, w1_ref[...], preferred_element_type=jnp.float32) \
        + b1_ref[...]
    z = _gelu(z)
    z = _ln(z, g_ref[...], bt_ref[...])
    o = jnp.dot(z, w2_ref[...], preferred_element_type=jnp.float32) \
        + b2_ref[...]
    m = jnp.max(o, axis=-1, keepdims=True)
    out_ref[...] = o - m - jnp.log(
        jnp.sum(jnp.exp(o - m), axis=-1, keepdims=True))


_GRID = NPAD // RB


def _row_spec(cols):
    if cols is None:
        return pl.BlockSpec((RB,), lambda i: (i,))
    return pl.BlockSpec((RB, cols), lambda i: (i, 0))


def _full_spec(shape):
    if len(shape) == 1:
        return pl.BlockSpec(shape, lambda i: (0,))
    return pl.BlockSpec(shape, lambda i: (0, 0))


def _tc_init(x, W_init, b_init, deg):
    return pl.pallas_call(
        _tc_init_body,
        grid=(_GRID,),
        in_specs=[_row_spec(D_IN), _full_spec((D_IN, H)), _full_spec((H,)),
                  _row_spec(None)],
        out_specs=[_row_spec(H), _row_spec(None)],
        out_shape=(jax.ShapeDtypeStruct((NPAD, H), jnp.float32),
                   jax.ShapeDtypeStruct((NPAD,), jnp.float32)),
    )(x, W_init, b_init, deg)


def _tc_layer(first, h_or_agg, zsp, hresp, dis, cb, g, bt, w):
    return pl.pallas_call(
        functools.partial(_tc_layer_body, first),
        grid=(_GRID,),
        in_specs=[_row_spec(H), _row_spec(H), _row_spec(H), _row_spec(None),
                  _full_spec((H,)), _full_spec((H,)), _full_spec((H,)),
                  _full_spec((H, H))],
        out_specs=[_row_spec(H), _row_spec(H)],
        out_shape=(jax.ShapeDtypeStruct((NPAD, H), jnp.float32),
                   jax.ShapeDtypeStruct((NPAD, H), jnp.float32)),
    )(h_or_agg, zsp, hresp, dis, cb, g, bt, w)


def _tc_final(agg, zsp, hresp, dis, cb, W_m1, b_m1, g, bt, W_m2, b_m2):
    return pl.pallas_call(
        _tc_final_body,
        grid=(_GRID,),
        in_specs=[_row_spec(H), _row_spec(H), _row_spec(H), _row_spec(None),
                  _full_spec((H,)), _full_spec((H, H)), _full_spec((H,)),
                  _full_spec((H,)), _full_spec((H,)), _full_spec((H, C)),
                  _full_spec((C,))],
        out_specs=[_row_spec(C)],
        out_shape=(jax.ShapeDtypeStruct((NPAD, C), jnp.float32),),
    )(agg, zsp, hresp, dis, cb, W_m1, b_m1, g, bt, W_m2, b_m2)


def kernel(x, edge_index, W_init, b_init, ln_gamma, ln_beta, conv_W, conv_b,
           W_m1, b_m1, mlp_ln_g, mlp_ln_b, W_m2, b_m2):
    deg, buckets, counts = _sc_bucketize(jnp.ravel(edge_index))
    xp = jnp.zeros((NPAD, D_IN), jnp.float32).at[:N].set(x)
    h0, dis = _tc_init(xp, W_init, b_init, deg)

    zs, hres = _tc_layer(True, h0, h0, h0, dis, conv_b[0],
                         ln_gamma[0], ln_beta[0], conv_W[0])
    for i in range(1, L):
        agg = _sc_aggregate(zs, buckets, counts)
        zs, hres = _tc_layer(False, agg, zs, hres, dis, conv_b[i - 1],
                             ln_gamma[i], ln_beta[i], conv_W[i])
    agg = _sc_aggregate(zs, buckets, counts)
    (out,) = _tc_final(agg, zs, hres, dis, conv_b[L - 1],
                       W_m1, b_m1, mlp_ln_g, mlp_ln_b, W_m2, b_m2)
    return out[:N]
